# baseline (speedup 1.0000x reference)
import functools

import jax
import jax.numpy as jnp
from jax.experimental import pallas as pl
from jax.experimental.pallas import tpu as pltpu

N = 1024
EC = 1024


def _bf(x):
    return x.astype(jnp.bfloat16)


def _hi_lo(x):
    hi = _bf(x)
    lo = _bf(x - hi.astype(jnp.float32))
    return hi, lo


def _leaky(z):
    return jnp.maximum(z, 0.2 * z)


def _iota(nrow):
    return jax.lax.broadcasted_iota(jnp.int32, (nrow, N), 1)


def _l1_kernel(x_ref, wl_ref, bl_ref, wr_ref, br_ref, att_ref, bias_ref,
               src_ref, dst_ref, h_ref, e_scr, *, n_chunks):
    x_bf = _bf(x_ref[...])
    xl = jnp.dot(x_bf, _bf(wl_ref[...]), preferred_element_type=jnp.float32) + bl_ref[...]
    xr = jnp.dot(x_bf, _bf(wr_ref[...]), preferred_element_type=jnp.float32) + br_ref[...]
    xl_hi, xl_lo = _hi_lo(xl)
    xr_hi, xr_lo = _hi_lo(xr)
    xlr_hi = jnp.concatenate([xl_hi, xr_hi], axis=0)
    xlr_lo = jnp.concatenate([xl_lo, xr_lo], axis=0)

    iota = _iota(EC)
    iota2 = jax.lax.broadcasted_iota(jnp.int32, (EC, 2 * N), 1)
    one = jnp.float32(1.0)
    zero = jnp.float32(0.0)
    for c in range(n_chunks):
        sl = slice(c * EC, (c + 1) * EC)
        psd = jnp.where((src_ref[sl, :] == iota2) | (dst_ref[sl, :] + N == iota2),
                        one, zero).astype(jnp.bfloat16)
        z = (jnp.dot(psd, xlr_hi, preferred_element_type=jnp.float32)
             + jnp.dot(psd, xlr_lo, preferred_element_type=jnp.float32))
        g = _leaky(z) * att_ref[...]
        e0 = jnp.sum(g[:, 0:8], axis=1, keepdims=True)
        e1 = jnp.sum(g[:, 8:16], axis=1, keepdims=True)
        e_scr[sl, :] = jnp.concatenate([e0, e1], axis=1)
    gmax = jnp.max(e_scr[...], axis=0, keepdims=True)

    dn = (((0,), (0,)), ((), ()))
    num = jnp.zeros((N, 16), jnp.float32)
    den = jnp.zeros((N, 2), jnp.float32)
    for c in range(n_chunks):
        sl = slice(c * EC, (c + 1) * EC)
        ps = jnp.where(src_ref[sl, :] == iota, one, zero).astype(jnp.bfloat16)
        pd = jnp.where(dst_ref[sl, :] == iota, one, zero).astype(jnp.bfloat16)
        xj = (jnp.dot(ps, xl_hi, preferred_element_type=jnp.float32)
              + jnp.dot(ps, xl_lo, preferred_element_type=jnp.float32))
        ex = jnp.exp(e_scr[sl, :] - gmax)
        exj = jnp.concatenate([xj[:, 0:8] * ex[:, 0:1], xj[:, 8:16] * ex[:, 1:2]], axis=1)
        exj_hi, exj_lo = _hi_lo(exj)
        ex_hi, ex_lo = _hi_lo(ex)
        num = num + (jax.lax.dot_general(pd, exj_hi, dn, preferred_element_type=jnp.float32)
                     + jax.lax.dot_general(pd, exj_lo, dn, preferred_element_type=jnp.float32))
        den = den + (jax.lax.dot_general(pd, ex_hi, dn, preferred_element_type=jnp.float32)
                     + jax.lax.dot_general(pd, ex_lo, dn, preferred_element_type=jnp.float32))
    r = 1.0 / den
    out = jnp.concatenate([num[:, 0:8] * r[:, 0:1], num[:, 8:16] * r[:, 1:2]], axis=1)
    h_ref[...] = out + bias_ref[...]


def _unit(xl, xr, att_row, src_ref, dst_ref, e_scr, mraw_ref, n_chunks):
    xl_hi, xl_lo = _hi_lo(xl)
    xr_hi = _bf(xr)
    xlr = jnp.concatenate([xl_hi, xr_hi], axis=0)
    iota = _iota(EC)
    iota2 = jax.lax.broadcasted_iota(jnp.int32, (EC, 2 * N), 1)
    one = jnp.float32(1.0)
    zero = jnp.float32(0.0)
    for c in range(n_chunks):
        sl = slice(c * EC, (c + 1) * EC)
        psd = jnp.where((src_ref[sl, :] == iota2) | (dst_ref[sl, :] + N == iota2),
                        one, zero).astype(jnp.bfloat16)
        z = jnp.dot(psd, xlr, preferred_element_type=jnp.float32)
        g = _leaky(z)
        e_scr[sl, :] = jnp.sum(g * att_row, axis=1, keepdims=True)
    gmax = jnp.max(e_scr[...])
    mraw_ref[...] = jnp.zeros((N, N), jnp.float32)
    dn = (((0,), (0,)), ((), ()))
    for c in range(n_chunks):
        sl = slice(c * EC, (c + 1) * EC)
        ex_c = jnp.exp(e_scr[sl, :] - gmax)
        ps = jnp.where(src_ref[sl, :] == iota, one, zero).astype(jnp.bfloat16)
        pdx = jnp.where(dst_ref[sl, :] == iota, ex_c, zero).astype(jnp.bfloat16)
        mraw_ref[...] += jax.lax.dot_general(pdx, ps, dn, preferred_element_type=jnp.float32)
    mraw_bf = _bf(mraw_ref[...])
    den = jnp.sum(mraw_bf.astype(jnp.float32), axis=1, keepdims=True)
    num = (jnp.dot(mraw_bf, xl_hi, preferred_element_type=jnp.float32)
           + jnp.dot(mraw_bf, xl_lo, preferred_element_type=jnp.float32))
    return num * (1.0 / den)


def _l2_kernel(h_ref, wl_ref, bl_ref, wr_ref, br_ref, att_ref, bias_ref,
               src_ref, dst_ref, hcat_ref, e_scr, mraw_ref, *, n_chunks):
    h_bf = _bf(h_ref[...])
    xl = jnp.dot(h_bf, _bf(wl_ref[0]), preferred_element_type=jnp.float32) + bl_ref[0]
    xr = jnp.dot(h_bf, _bf(wr_ref[0]), preferred_element_type=jnp.float32) + br_ref[0]
    out = _unit(xl, xr, att_ref[0], src_ref, dst_ref, e_scr, mraw_ref, n_chunks)
    hcat_ref[...] = _bf(out + bias_ref[0])


def _l3_kernel(hcat_ref, wl_ref, wr_ref, bl_ref, br_ref, att_ref, bias_ref,
               src_ref, dst_ref, wout_ref, bout_ref, out_ref,
               accl_ref, accr_ref, e_scr, mraw_ref, *, n_chunks, kt):
    k = pl.program_id(1)
    h = pl.program_id(0)

    @pl.when(k == 0)
    def _():
        accl_ref[...] = jnp.zeros_like(accl_ref)
        accr_ref[...] = jnp.zeros_like(accr_ref)

    hc = hcat_ref[...]
    accl_ref[...] += jnp.dot(hc, _bf(wl_ref[...]), preferred_element_type=jnp.float32)
    accr_ref[...] += jnp.dot(hc, _bf(wr_ref[...]), preferred_element_type=jnp.float32)

    @pl.when(k == kt - 1)
    def _():
        xl = accl_ref[...] + bl_ref[0]
        xr = accr_ref[...] + br_ref[0]
        g = _unit(xl, xr, att_ref[0], src_ref, dst_ref, e_scr, mraw_ref, n_chunks)
        g_bf = _bf(g + bias_ref[0])
        val = jnp.dot(g_bf, _bf(wout_ref[...]), preferred_element_type=jnp.float32)

        @pl.when(h == 0)
        def _():
            out_ref[...] = val + bout_ref[...]

        @pl.when(h > 0)
        def _():
            out_ref[...] += val


def kernel(x, edge_index, params):
    n = x.shape[0]
    e_tot = edge_index.shape[1] + n
    n_chunks = e_tot // EC
    loop = jnp.arange(n, dtype=edge_index.dtype)
    ei = jnp.concatenate([edge_index, jnp.stack([loop, loop])], axis=1)
    src = ei[0].reshape(e_tot, 1)
    dst = ei[1].reshape(e_tot, 1)

    p1 = params["init_conv"]
    h = pl.pallas_call(
        functools.partial(_l1_kernel, n_chunks=n_chunks),
        out_shape=jax.ShapeDtypeStruct((n, 16), jnp.float32),
        scratch_shapes=[pltpu.VMEM((e_tot, 2), jnp.float32)],
        compiler_params=pltpu.CompilerParams(
            fuse_transposed_lhs_in_matmul=True),
    )(x, p1["Wl"], p1["bl"].reshape(1, 16), p1["Wr"], p1["br"].reshape(1, 16),
      p1["att"].reshape(1, 16), p1["bias"].reshape(1, 16), src, dst)

    hc = params["head_convs"]
    wl_st = jnp.stack([p["Wl"] for p in hc])
    bl_st = jnp.stack([p["bl"] for p in hc])
    wr_st = jnp.stack([p["Wr"] for p in hc])
    br_st = jnp.stack([p["br"] for p in hc])
    att_st = jnp.stack([p["att"][0] for p in hc])
    bias_st = jnp.stack([p["bias"] for p in hc])

    full_edge = pl.BlockSpec((e_tot, 1), lambda i: (0, 0))
    hcat = pl.pallas_call(
        functools.partial(_l2_kernel, n_chunks=n_chunks),
        grid=(16,),
        in_specs=[
            pl.BlockSpec((n, 16), lambda i: (0, 0)),
            pl.BlockSpec((1, 16, 1024), lambda i: (i, 0, 0)),
            pl.BlockSpec((1, 1, 1024), lambda i: (i, 0, 0)),
            pl.BlockSpec((1, 16, 1024), lambda i: (i, 0, 0)),
            pl.BlockSpec((1, 1, 1024), lambda i: (i, 0, 0)),
            pl.BlockSpec((1, 1, 1024), lambda i: (i, 0, 0)),
            pl.BlockSpec((1, 1, 1024), lambda i: (i, 0, 0)),
            full_edge,
            full_edge,
        ],
        out_specs=pl.BlockSpec((n, 1024), lambda i: (0, i)),
        out_shape=jax.ShapeDtypeStruct((n, 16 * 1024), jnp.bfloat16),
        scratch_shapes=[pltpu.VMEM((e_tot, 1), jnp.float32),
                        pltpu.VMEM((n, N), jnp.float32)],
        compiler_params=pltpu.CompilerParams(
            dimension_semantics=("arbitrary",),
            fuse_transposed_lhs_in_matmul=True),
    )(h, wl_st, bl_st.reshape(16, 1, 1024), wr_st, br_st.reshape(16, 1, 1024),
      att_st.reshape(16, 1, 1024), bias_st.reshape(16, 1, 1024), src, dst)

    pg = params["gat"]
    po = params["out"]
    kt = 32
    kb = 16384 // kt
    full_edge2 = pl.BlockSpec((e_tot, 1), lambda hh, kk: (0, 0))
    out = pl.pallas_call(
        functools.partial(_l3_kernel, n_chunks=n_chunks, kt=kt),
        grid=(16, kt),
        in_specs=[
            pl.BlockSpec((n, kb), lambda hh, kk: (0, kk)),
            pl.BlockSpec((kb, 1024), lambda hh, kk: (kk, hh)),
            pl.BlockSpec((kb, 1024), lambda hh, kk: (kk, hh)),
            pl.BlockSpec((1, 1, 1024), lambda hh, kk: (hh, 0, 0)),
            pl.BlockSpec((1, 1, 1024), lambda hh, kk: (hh, 0, 0)),
            pl.BlockSpec((1, 1, 1024), lambda hh, kk: (hh, 0, 0)),
            pl.BlockSpec((1, 1, 1024), lambda hh, kk: (hh, 0, 0)),
            full_edge2,
            full_edge2,
            pl.BlockSpec((1024, 64), lambda hh, kk: (hh, 0)),
            pl.BlockSpec((1, 64), lambda hh, kk: (0, 0)),
        ],
        out_specs=pl.BlockSpec((n, 64), lambda hh, kk: (0, 0)),
        out_shape=jax.ShapeDtypeStruct((n, 64), jnp.float32),
        scratch_shapes=[
            pltpu.VMEM((n, 1024), jnp.float32),
            pltpu.VMEM((n, 1024), jnp.float32),
            pltpu.VMEM((e_tot, 1), jnp.float32),
            pltpu.VMEM((n, N), jnp.float32),
        ],
        compiler_params=pltpu.CompilerParams(
            dimension_semantics=("arbitrary", "arbitrary"),
            fuse_transposed_lhs_in_matmul=True),
    )(hcat, pg["Wl"], pg["Wr"], pg["bl"].reshape(16, 1, 1024),
      pg["br"].reshape(16, 1, 1024), pg["att"].reshape(16, 1, 1024),
      pg["bias"].reshape(16, 1, 1024), src, dst, po["W"], po["b"].reshape(1, 64))
    return out


# speedup vs baseline: 1.2391x; 1.2391x over previous
import functools

import jax
import jax.numpy as jnp
from jax.experimental import pallas as pl
from jax.experimental.pallas import tpu as pltpu

N = 1024
EC = 1024


def _bf(x):
    return x.astype(jnp.bfloat16)


def _hi_lo(x):
    hi = _bf(x)
    lo = _bf(x - hi.astype(jnp.float32))
    return hi, lo


def _leaky(z):
    return jnp.maximum(z, 0.2 * z)


def _sd(comb_ref, c):
    comb = comb_ref[:, c:c + 1]
    return jnp.bitwise_and(comb, N - 1), jnp.right_shift(comb, 10)


def _psd(comb_ref, c, iota2):
    src_c, dst_c = _sd(comb_ref, c)
    one = jnp.float32(1.0)
    zero = jnp.float32(0.0)
    return jnp.where((src_c == iota2) | (dst_c + N == iota2), one, zero
                     ).astype(jnp.bfloat16)


def _l1_kernel(x_ref, wl_ref, bl_ref, wr_ref, br_ref, att_ref, bias_ref,
               comb_ref, h_ref, e_scr, *, n_chunks):
    x_bf = _bf(x_ref[...])
    xl = jnp.dot(x_bf, _bf(wl_ref[...]), preferred_element_type=jnp.float32) + bl_ref[...]
    xr = jnp.dot(x_bf, _bf(wr_ref[...]), preferred_element_type=jnp.float32) + br_ref[...]
    xl_hi, xl_lo = _hi_lo(xl)
    xr_hi, xr_lo = _hi_lo(xr)
    xlr_hi = jnp.concatenate([xl_hi, xr_hi], axis=0)
    xlr_lo = jnp.concatenate([xl_lo, xr_lo], axis=0)

    iota = jax.lax.broadcasted_iota(jnp.int32, (EC, N), 1)
    iota2 = jax.lax.broadcasted_iota(jnp.int32, (EC, 2 * N), 1)
    one = jnp.float32(1.0)
    zero = jnp.float32(0.0)
    for c in range(n_chunks):
        psd = _psd(comb_ref, c, iota2)
        z = (jnp.dot(psd, xlr_hi, preferred_element_type=jnp.float32)
             + jnp.dot(psd, xlr_lo, preferred_element_type=jnp.float32))
        g = _leaky(z) * att_ref[...]
        e_scr[:, c:c + 1] = jnp.sum(g[:, 0:8], axis=1, keepdims=True)
        e_scr[:, n_chunks + c:n_chunks + c + 1] = jnp.sum(g[:, 8:16], axis=1, keepdims=True)
    gmax0 = jnp.max(e_scr[:, 0:n_chunks])
    gmax1 = jnp.max(e_scr[:, n_chunks:2 * n_chunks])

    dn = (((0,), (0,)), ((), ()))
    num = jnp.zeros((N, 16), jnp.float32)
    den = jnp.zeros((N, 2), jnp.float32)
    for c in range(n_chunks):
        src_c, dst_c = _sd(comb_ref, c)
        ps = jnp.where(src_c == iota, one, zero).astype(jnp.bfloat16)
        pd = jnp.where(dst_c == iota, one, zero).astype(jnp.bfloat16)
        xj = (jnp.dot(ps, xl_hi, preferred_element_type=jnp.float32)
              + jnp.dot(ps, xl_lo, preferred_element_type=jnp.float32))
        ex0 = jnp.exp(e_scr[:, c:c + 1] - gmax0)
        ex1 = jnp.exp(e_scr[:, n_chunks + c:n_chunks + c + 1] - gmax1)
        ex = jnp.concatenate([ex0, ex1], axis=1)
        exj = jnp.concatenate([xj[:, 0:8] * ex0, xj[:, 8:16] * ex1], axis=1)
        exj_hi, exj_lo = _hi_lo(exj)
        ex_hi, ex_lo = _hi_lo(ex)
        num = num + (jax.lax.dot_general(pd, exj_hi, dn, preferred_element_type=jnp.float32)
                     + jax.lax.dot_general(pd, exj_lo, dn, preferred_element_type=jnp.float32))
        den = den + (jax.lax.dot_general(pd, ex_hi, dn, preferred_element_type=jnp.float32)
                     + jax.lax.dot_general(pd, ex_lo, dn, preferred_element_type=jnp.float32))
    r = 1.0 / den
    out = jnp.concatenate([num[:, 0:8] * r[:, 0:1], num[:, 8:16] * r[:, 1:2]], axis=1)
    h_ref[...] = out + bias_ref[...]


def _unit(xl, xr, att_row, comb_ref, e_scr, mraw_ref, n_chunks):
    xl_hi, xl_lo = _hi_lo(xl)
    xr_hi = _bf(xr)
    xlr = jnp.concatenate([xl_hi, xr_hi], axis=0)
    iota = jax.lax.broadcasted_iota(jnp.int32, (EC, N), 1)
    iota2 = jax.lax.broadcasted_iota(jnp.int32, (EC, 2 * N), 1)
    one = jnp.float32(1.0)
    zero = jnp.float32(0.0)
    for c in range(n_chunks):
        psd = _psd(comb_ref, c, iota2)
        z = jnp.dot(psd, xlr, preferred_element_type=jnp.float32)
        g = _leaky(z)
        e_scr[:, c:c + 1] = jnp.sum(g * att_row, axis=1, keepdims=True)
    gmax = jnp.max(e_scr[...])
    mraw_ref[...] = jnp.zeros((N, N), jnp.float32)
    dn = (((0,), (0,)), ((), ()))
    for c in range(n_chunks):
        ex_c = jnp.exp(e_scr[:, c:c + 1] - gmax)
        src_c, dst_c = _sd(comb_ref, c)
        ps = jnp.where(src_c == iota, one, zero).astype(jnp.bfloat16)
        pdx = jnp.where(dst_c == iota, ex_c, zero).astype(jnp.bfloat16)
        mraw_ref[...] += jax.lax.dot_general(pdx, ps, dn, preferred_element_type=jnp.float32)
    mraw_bf = _bf(mraw_ref[...])
    den = jnp.sum(mraw_bf.astype(jnp.float32), axis=1, keepdims=True)
    num = (jnp.dot(mraw_bf, xl_hi, preferred_element_type=jnp.float32)
           + jnp.dot(mraw_bf, xl_lo, preferred_element_type=jnp.float32))
    return num * (1.0 / den)


def _l2_kernel(h_ref, wl_ref, bl_ref, wr_ref, br_ref, att_ref, bias_ref,
               comb_ref, hcat_ref, e_scr, mraw_ref, *, n_chunks):
    h_bf = _bf(h_ref[...])
    xl = jnp.dot(h_bf, _bf(wl_ref[0]), preferred_element_type=jnp.float32) + bl_ref[0]
    xr = jnp.dot(h_bf, _bf(wr_ref[0]), preferred_element_type=jnp.float32) + br_ref[0]
    out = _unit(xl, xr, att_ref[0], comb_ref, e_scr, mraw_ref, n_chunks)
    hcat_ref[...] = _bf(out + bias_ref[0])


def _l3_kernel(hcat_ref, wl_ref, wr_ref, bl_ref, br_ref, att_ref, bias_ref,
               comb_ref, wout_ref, bout_ref, out_ref,
               accl_ref, accr_ref, e_scr, mraw_ref, *, n_chunks, kt):
    k = pl.program_id(1)
    h = pl.program_id(0)

    @pl.when(k == 0)
    def _():
        accl_ref[...] = jnp.zeros_like(accl_ref)
        accr_ref[...] = jnp.zeros_like(accr_ref)

    hc = hcat_ref[...]
    accl_ref[...] += jnp.dot(hc, _bf(wl_ref[...]), preferred_element_type=jnp.float32)
    accr_ref[...] += jnp.dot(hc, _bf(wr_ref[...]), preferred_element_type=jnp.float32)

    @pl.when(k == kt - 1)
    def _():
        xl = accl_ref[...] + bl_ref[0]
        xr = accr_ref[...] + br_ref[0]
        g = _unit(xl, xr, att_ref[0], comb_ref, e_scr, mraw_ref, n_chunks)
        g_bf = _bf(g + bias_ref[0])
        val = jnp.dot(g_bf, _bf(wout_ref[...]), preferred_element_type=jnp.float32)

        @pl.when(h == 0)
        def _():
            out_ref[...] = val + bout_ref[...]

        @pl.when(h > 0)
        def _():
            out_ref[...] += val


def kernel(x, edge_index, params):
    n = x.shape[0]
    e_tot = edge_index.shape[1] + n
    n_chunks = e_tot // EC
    loop = jnp.arange(n, dtype=edge_index.dtype)
    ei = jnp.concatenate([edge_index, jnp.stack([loop, loop])], axis=1)
    comb = (ei[0] + ei[1] * n).astype(jnp.int32)
    comb = comb.reshape(n_chunks, EC).T

    p1 = params["init_conv"]
    h = pl.pallas_call(
        functools.partial(_l1_kernel, n_chunks=n_chunks),
        out_shape=jax.ShapeDtypeStruct((n, 16), jnp.float32),
        scratch_shapes=[pltpu.VMEM((EC, 2 * n_chunks), jnp.float32)],
    )(x, p1["Wl"], p1["bl"].reshape(1, 16), p1["Wr"], p1["br"].reshape(1, 16),
      p1["att"].reshape(1, 16), p1["bias"].reshape(1, 16), comb)

    hc = params["head_convs"]
    wl_st = jnp.stack([p["Wl"] for p in hc])
    bl_st = jnp.stack([p["bl"] for p in hc])
    wr_st = jnp.stack([p["Wr"] for p in hc])
    br_st = jnp.stack([p["br"] for p in hc])
    att_st = jnp.stack([p["att"][0] for p in hc])
    bias_st = jnp.stack([p["bias"] for p in hc])

    comb_spec = pl.BlockSpec((EC, n_chunks), lambda i: (0, 0))
    hcat = pl.pallas_call(
        functools.partial(_l2_kernel, n_chunks=n_chunks),
        grid=(16,),
        in_specs=[
            pl.BlockSpec((n, 16), lambda i: (0, 0)),
            pl.BlockSpec((1, 16, 1024), lambda i: (i, 0, 0)),
            pl.BlockSpec((1, 1, 1024), lambda i: (i, 0, 0)),
            pl.BlockSpec((1, 16, 1024), lambda i: (i, 0, 0)),
            pl.BlockSpec((1, 1, 1024), lambda i: (i, 0, 0)),
            pl.BlockSpec((1, 1, 1024), lambda i: (i, 0, 0)),
            pl.BlockSpec((1, 1, 1024), lambda i: (i, 0, 0)),
            comb_spec,
        ],
        out_specs=pl.BlockSpec((n, 1024), lambda i: (0, i)),
        out_shape=jax.ShapeDtypeStruct((n, 16 * 1024), jnp.bfloat16),
        scratch_shapes=[pltpu.VMEM((EC, n_chunks), jnp.float32),
                        pltpu.VMEM((n, N), jnp.float32)],
        compiler_params=pltpu.CompilerParams(
            dimension_semantics=("arbitrary",)),
    )(h, wl_st, bl_st.reshape(16, 1, 1024), wr_st, br_st.reshape(16, 1, 1024),
      att_st.reshape(16, 1, 1024), bias_st.reshape(16, 1, 1024), comb)

    pg = params["gat"]
    po = params["out"]
    kt = 16
    kb = 16384 // kt
    comb_spec2 = pl.BlockSpec((EC, n_chunks), lambda hh, kk: (0, 0))
    out = pl.pallas_call(
        functools.partial(_l3_kernel, n_chunks=n_chunks, kt=kt),
        grid=(16, kt),
        in_specs=[
            pl.BlockSpec((n, kb), lambda hh, kk: (0, kk)),
            pl.BlockSpec((kb, 1024), lambda hh, kk: (kk, hh)),
            pl.BlockSpec((kb, 1024), lambda hh, kk: (kk, hh)),
            pl.BlockSpec((1, 1, 1024), lambda hh, kk: (hh, 0, 0)),
            pl.BlockSpec((1, 1, 1024), lambda hh, kk: (hh, 0, 0)),
            pl.BlockSpec((1, 1, 1024), lambda hh, kk: (hh, 0, 0)),
            pl.BlockSpec((1, 1, 1024), lambda hh, kk: (hh, 0, 0)),
            comb_spec2,
            pl.BlockSpec((1024, 64), lambda hh, kk: (hh, 0)),
            pl.BlockSpec((1, 64), lambda hh, kk: (0, 0)),
        ],
        out_specs=pl.BlockSpec((n, 64), lambda hh, kk: (0, 0)),
        out_shape=jax.ShapeDtypeStruct((n, 64), jnp.float32),
        scratch_shapes=[
            pltpu.VMEM((n, 1024), jnp.float32),
            pltpu.VMEM((n, 1024), jnp.float32),
            pltpu.VMEM((EC, n_chunks), jnp.float32),
            pltpu.VMEM((n, N), jnp.float32),
        ],
        compiler_params=pltpu.CompilerParams(
            dimension_semantics=("arbitrary", "arbitrary")),
    )(hcat, pg["Wl"], pg["Wr"], pg["bl"].reshape(16, 1, 1024),
      pg["br"].reshape(16, 1, 1024), pg["att"].reshape(16, 1, 1024),
      pg["bias"].reshape(16, 1, 1024), comb, po["W"], po["b"].reshape(1, 64))
    return out
